# baseline (speedup 1.0000x reference)
import functools

import jax
import jax.numpy as jnp
from jax import lax
from jax.experimental import pallas as pl
from jax.experimental.pallas import tpu as pltpu
from jax.experimental.pallas import tpu_sc as plsc

B = 4
S = 4096
D = 768
N_ROWS = B * S
NC = 2
NS = 16
NW = NC * NS
S_PER_W = S // NW
CH = 16
N_CC = S_PER_W // CH
N_G = N_CC * 2
LANES = 16
D_VECS = D // LANES
SCALE = 27.712812921102035


def _sc_body(x_hbm, pe_hbm, table_hbm, out_hbm,
             idx_v, rows_all, pes, gsem, psem, ssem):
    wid = lax.axis_index("s") * NC + lax.axis_index("c")
    w0 = wid * S_PER_W

    for b in range(B):
        pltpu.sync_copy(x_hbm.at[b, pl.ds(w0, S_PER_W)],
                        idx_v.at[pl.ds(b * S_PER_W, S_PER_W)])

    def gather_desc(t, sub):
        cc = t // 2
        bp = t % 2
        slot = t % 3
        b = 2 * bp + sub
        ioff = b * S_PER_W + cc * CH
        return pltpu.make_async_copy(
            table_hbm.at[idx_v.at[pl.ds(ioff, CH)]],
            rows_all.at[pl.ds(slot * 2 * CH + sub * CH, CH)],
            gsem.at[slot])

    def store_desc(t, sub):
        cc = t // 2
        bp = t % 2
        slot = t % 3
        b = 2 * bp + sub
        return pltpu.make_async_copy(
            rows_all.at[pl.ds(slot * 2 * CH + sub * CH, CH)],
            out_hbm.at[pl.ds(b * S + w0 + cc * CH, CH)],
            ssem.at[slot])

    def pe_desc(cc):
        return pltpu.make_async_copy(
            pe_hbm.at[pl.ds(w0 + cc * CH, CH)],
            pes.at[pl.ds((cc % 2) * CH, CH)],
            psem.at[cc % 2])

    for t in (0, 1):
        gather_desc(t, 0).start()
        gather_desc(t, 1).start()
    pe_desc(0).start()

    def group_body(t, _):
        cc = t // 2
        bp = t % 2
        slot = t % 3

        @pl.when(bp == 0)
        def _():
            pe_desc(cc).wait()

        gather_desc(t, 0).wait()
        gather_desc(t, 1).wait()

        rbase = slot * 2 * CH
        pbase = (cc % 2) * CH

        def row_body(r, _):
            for d in range(D_VECS):
                sl = pl.ds(d * LANES, LANES)
                pv = pes[pbase + r, sl]
                rows_all[rbase + r, sl] = rows_all[rbase + r, sl] * SCALE + pv
                rows_all[rbase + CH + r, sl] = (
                    rows_all[rbase + CH + r, sl] * SCALE + pv)
            return 0

        lax.fori_loop(0, CH, row_body, 0)

        store_desc(t, 0).start()
        store_desc(t, 1).start()

        @pl.when(t + 2 < N_G)
        def _():
            @pl.when(t >= 1)
            def _():
                store_desc(t - 1, 0).wait()
                store_desc(t - 1, 1).wait()
            gather_desc(t + 2, 0).start()
            gather_desc(t + 2, 1).start()

        @pl.when(jnp.logical_and(bp == 1, cc + 1 < N_CC))
        def _():
            pe_desc(cc + 1).start()

        return 0

    lax.fori_loop(0, N_G, group_body, 0)

    for t in range(N_G - 3, N_G):
        store_desc(t, 0).wait()
        store_desc(t, 1).wait()


@jax.jit
def _embed(x, pe, table):
    mesh = plsc.VectorSubcoreMesh(core_axis_name="c", subcore_axis_name="s")
    k = functools.partial(
        pl.kernel,
        mesh=mesh,
        out_type=jax.ShapeDtypeStruct((N_ROWS, D), jnp.float32),
        scratch_types=[
            pltpu.VMEM((B * S_PER_W,), jnp.int32),
            pltpu.VMEM((3 * 2 * CH, D), jnp.float32),
            pltpu.VMEM((2 * CH, D), jnp.float32),
            pltpu.SemaphoreType.DMA((3,)),
            pltpu.SemaphoreType.DMA((2,)),
            pltpu.SemaphoreType.DMA((3,)),
        ],
    )(_sc_body)
    return k(x, pe, table)


def kernel(x, token_table, pe):
    out = _embed(x.astype(jnp.int32), pe, token_table)
    return out.reshape(B, S, D)


# speedup vs baseline: 2.1448x; 2.1448x over previous
import functools

import jax
import jax.numpy as jnp
from jax import lax
from jax.experimental import pallas as pl
from jax.experimental.pallas import tpu as pltpu
from jax.experimental.pallas import tpu_sc as plsc

B = 4
S = 4096
D = 768
N_ROWS = B * S
NC = 2
NS = 16
NW = NC * NS
S_PER_W = S // NW
CH = 16
LANES = 16
D_VECS = D // LANES
SCALE = 27.712812921102035
N_K = 4


def _sc_body(x_hbm, pe_hbm, table_hbm, out_hbm,
             idx_v, rA, rB, rC, rD, peb,
             g0, g1, g2, g3, p0, p1, s0_, s1_, s2_, s3_):
    rows = [rA, rB, rC, rD]
    gsem = [g0, g1, g2, g3]
    psem = [p0, p1]
    ssem = [s0_, s1_, s2_, s3_]

    wid = lax.axis_index("s") * NC + lax.axis_index("c")
    w0 = wid * S_PER_W

    for b in range(B):
        pltpu.sync_copy(x_hbm.at[b, pl.ds(w0, S_PER_W)],
                        idx_v.at[pl.ds(b * S_PER_W, S_PER_W)])

    def gdesc(slot, k, j, sub):
        cc = 2 * k + j // 2
        b = 2 * (j % 2) + sub
        return pltpu.make_async_copy(
            table_hbm.at[idx_v.at[pl.ds(b * S_PER_W + cc * CH, CH)]],
            rows[slot].at[pl.ds(sub * CH, CH)], gsem[slot])

    def sdesc(slot, k, j, sub):
        cc = 2 * k + j // 2
        b = 2 * (j % 2) + sub
        return pltpu.make_async_copy(
            rows[slot].at[pl.ds(sub * CH, CH)],
            out_hbm.at[pl.ds(b * S + w0 + cc * CH, CH)], ssem[slot])

    def pdesc(half, cc):
        return pltpu.make_async_copy(
            pe_hbm.at[pl.ds(w0 + cc * CH, CH)],
            peb.at[pl.ds(half * CH, CH)], psem[half])

    for j in (0, 1):
        gdesc(j, 0, j, 0).start()
        gdesc(j, 0, j, 1).start()
    pdesc(0, 0).start()
    pdesc(1, 1).start()

    def k_body(k, _):
        for j in range(4):
            half = j // 2
            if j == 0:
                pdesc(0, 2 * k).wait()
            if j == 2:
                pdesc(1, 2 * k + 1).wait()
            gdesc(j, k, j, 0).wait()
            gdesc(j, k, j, 1).wait()

            def row_body(r, _, _j=j, _half=half):
                rr = rows[_j]
                for d in range(D_VECS):
                    sl = pl.ds(d * LANES, LANES)
                    pv = peb[_half * CH + r, sl]
                    rr[r, sl] = rr[r, sl] * SCALE + pv
                    rr[CH + r, sl] = rr[CH + r, sl] * SCALE + pv
                return 0

            lax.fori_loop(0, CH, row_body, 0)

            sdesc(j, k, j, 0).start()
            sdesc(j, k, j, 1).start()

            if j < 2:
                @pl.when(k >= 1)
                def _(_j=j):
                    sdesc(_j + 2, k - 1, _j + 2, 0).wait()
                    sdesc(_j + 2, k - 1, _j + 2, 1).wait()
                gdesc(j + 2, k, j + 2, 0).start()
                gdesc(j + 2, k, j + 2, 1).start()
            else:
                @pl.when(k < N_K - 1)
                def _(_j=j):
                    sdesc(_j - 2, k, _j - 2, 0).wait()
                    sdesc(_j - 2, k, _j - 2, 1).wait()
                    gdesc(_j - 2, k + 1, _j - 2, 0).start()
                    gdesc(_j - 2, k + 1, _j - 2, 1).start()
            if j == 1:
                @pl.when(k < N_K - 1)
                def _():
                    pdesc(0, 2 * k + 2).start()
            if j == 3:
                @pl.when(k < N_K - 1)
                def _():
                    pdesc(1, 2 * k + 3).start()
        return 0

    lax.fori_loop(0, N_K, k_body, 0)

    for j in range(4):
        sdesc(j, N_K - 1, j, 0).wait()
        sdesc(j, N_K - 1, j, 1).wait()


@jax.jit
def _embed(x, pe, table):
    mesh = plsc.VectorSubcoreMesh(core_axis_name="c", subcore_axis_name="s")
    k = functools.partial(
        pl.kernel,
        mesh=mesh,
        out_type=jax.ShapeDtypeStruct((N_ROWS, D), jnp.float32),
        scratch_types=[
            pltpu.VMEM((B * S_PER_W,), jnp.int32),
            pltpu.VMEM((2 * CH, D), jnp.float32),
            pltpu.VMEM((2 * CH, D), jnp.float32),
            pltpu.VMEM((2 * CH, D), jnp.float32),
            pltpu.VMEM((2 * CH, D), jnp.float32),
            pltpu.VMEM((2 * CH, D), jnp.float32),
            pltpu.SemaphoreType.DMA,
            pltpu.SemaphoreType.DMA,
            pltpu.SemaphoreType.DMA,
            pltpu.SemaphoreType.DMA,
            pltpu.SemaphoreType.DMA,
            pltpu.SemaphoreType.DMA,
            pltpu.SemaphoreType.DMA,
            pltpu.SemaphoreType.DMA,
            pltpu.SemaphoreType.DMA,
            pltpu.SemaphoreType.DMA,
        ],
    )(_sc_body)
    return k(x, pe, table)


def kernel(x, token_table, pe):
    out = _embed(x.astype(jnp.int32), pe, token_table)
    return out.reshape(B, S, D)
